# Initial kernel scaffold; baseline (speedup 1.0000x reference)
import jax
import jax.numpy as jnp
from jax.experimental import pallas as pl


def kernel(x, edge_index, edge_type, batch, W0, W1, W2, g0, b0, g1, b1, g2, b2):
    raise NotImplementedError("write your pallas kernel here")


# speedup vs baseline: 10.7787x; 10.7787x over previous
import functools

import jax
import jax.numpy as jnp
from jax import lax
from jax.experimental import pallas as pl
from jax.experimental.pallas import tpu as pltpu
from jax.experimental.pallas import tpu_sc as plsc

_NUM_GRAPHS = 16
_EPS = 1e-5
_NC = 2
_NS = 16
_CHUNK = 128


def _mm_body(z_ref, w_ref, h_ref):
    h_ref[0] = jnp.maximum(
        lax.dot_general(z_ref[...], w_ref[0], (((1,), (0,)), ((), ())),
                        preferred_element_type=jnp.float32),
        0.0)


def _relu_mm(zin, W, bm):
    npad, d = zin.shape
    num_rel, _, hid = W.shape
    return pl.pallas_call(
        _mm_body,
        grid=(npad // bm, num_rel),
        in_specs=[
            pl.BlockSpec((bm, d), lambda i, r: (i, 0)),
            pl.BlockSpec((1, d, hid), lambda i, r: (r, 0, 0)),
        ],
        out_specs=pl.BlockSpec((1, bm, hid), lambda i, r: (r, i, 0)),
        out_shape=jax.ShapeDtypeStruct((num_rel, npad, hid), jnp.float32),
    )(zin, W)


def _make_edge_kernel(npad, hid, chunks_per_worker, nrows_tab):
    rows_per_tile = npad // _NS

    mesh = plsc.VectorSubcoreMesh(core_axis_name="c", subcore_axis_name="s")

    @functools.partial(
        pl.kernel, mesh=mesh,
        out_type=jax.ShapeDtypeStruct((_NC, npad, hid), jnp.float32),
        scratch_types=[
            pltpu.VMEM((chunks_per_worker, _CHUNK), jnp.int32),
            pltpu.VMEM((chunks_per_worker, _CHUNK), jnp.int32),
            pltpu.VMEM((_CHUNK, hid), jnp.float32),
            pltpu.VMEM_SHARED((npad, hid), jnp.float32),
        ],
    )
    def edge_kernel(h_hbm, gidx_hbm, src_hbm, out_hbm, gidx_v, src_v, rows_v,
                    acc_sh):
        c = lax.axis_index("c")
        s = lax.axis_index("s")
        wid = c * _NS + s
        row_base = wid * chunks_per_worker

        pltpu.sync_copy(gidx_hbm.at[pl.ds(row_base, chunks_per_worker)], gidx_v)
        pltpu.sync_copy(src_hbm.at[pl.ds(row_base, chunks_per_worker)], src_v)

        def _zero_body(i, carry):
            r = i // (hid // 16)
            col = (i % (hid // 16)) * 16
            rows_v[r, pl.ds(col, 16)] = jnp.zeros((16,), jnp.float32)
            return carry
        lax.fori_loop(0, _CHUNK * (hid // 16), _zero_body, 0)
        for t in range(rows_per_tile // _CHUNK):
            pltpu.sync_copy(
                rows_v, acc_sh.at[pl.ds(s * rows_per_tile + t * _CHUNK, _CHUNK)])
        plsc.subcore_barrier()

        def _body(k, carry):
            pltpu.sync_copy(h_hbm.at[gidx_v.at[k]], rows_v)
            pltpu.sync_copy(rows_v, acc_sh.at[src_v.at[k]], add=True)
            return carry
        lax.fori_loop(0, chunks_per_worker, _body, 0)
        plsc.subcore_barrier()

        for t in range(rows_per_tile // _CHUNK):
            off = s * rows_per_tile + t * _CHUNK
            pltpu.sync_copy(acc_sh.at[pl.ds(off, _CHUNK)], rows_v)
            pltpu.sync_copy(rows_v, out_hbm.at[c, pl.ds(off, _CHUNK)])

    return edge_kernel


def _make_bn(n, npad, hid):
    def _bn_body(p_ref, g_ref, b_ref, z_ref):
        o = p_ref[0, :n, :] + p_ref[1, :n, :]
        mean = jnp.mean(o, axis=0, keepdims=True)
        d = o - mean
        var = jnp.mean(d * d, axis=0, keepdims=True)
        zn = d * lax.rsqrt(var + _EPS) * g_ref[...] + b_ref[...]
        z_ref[:n, :] = jnp.maximum(zn, 0.0)
        z_ref[n:, :] = jnp.zeros((npad - n, hid), jnp.float32)

    return pl.pallas_call(
        _bn_body,
        out_shape=jax.ShapeDtypeStruct((npad, hid), jnp.float32),
    )


def _make_seg(n, dim):
    def _seg_body(b_ref, z_ref, g_ref):
        ids = jnp.broadcast_to(b_ref[...], (_NUM_GRAPHS, n))
        onehot = (ids == lax.broadcasted_iota(jnp.int32, (_NUM_GRAPHS, n), 0)
                  ).astype(jnp.float32)
        g_ref[...] = lax.dot_general(
            onehot, z_ref[...], (((1,), (0,)), ((), ())),
            preferred_element_type=jnp.float32)

    return pl.pallas_call(
        _seg_body,
        out_shape=jax.ShapeDtypeStruct((_NUM_GRAPHS, dim), jnp.float32),
    )


def kernel(x, edge_index, edge_type, batch, W0, W1, W2, g0, b0, g1, b1, g2,
           b2):
    n, in_dim = x.shape
    ne = edge_index.shape[1]
    num_rel, _, hid = W0.shape
    id_dim = W1.shape[1] - hid

    nw = _NC * _NS
    npad = ((n + 2047) // 2048) * 2048
    epw = ((ne + nw * _CHUNK * 8 - 1) // (nw * _CHUNK * 8)) * _CHUNK * 8
    ne_pad = epw * nw
    chunks_per_worker = epw // _CHUNK

    src = edge_index[0].astype(jnp.int32)
    dst = edge_index[1].astype(jnp.int32)
    et = edge_type.astype(jnp.int32)
    pad = ne_pad - ne
    gidx = jnp.concatenate(
        [et * npad + dst, jnp.zeros((pad,), jnp.int32)]).reshape(-1, _CHUNK)
    srcp = jnp.concatenate(
        [src, jnp.full((pad,), npad - 1, jnp.int32)]).reshape(-1, _CHUNK)

    x_pad = jnp.pad(x, ((0, npad - n), (0, 0)))
    edge_call = _make_edge_kernel(npad, hid, chunks_per_worker,
                                  num_rel * npad)
    bn_call = _make_bn(n, npad, hid)

    Ws = (W0, W1, W2)
    gs = (g0, g1, g2)
    bs = (b0, b1, b2)
    zin = x_pad
    zs = []
    for l in range(3):
        H = _relu_mm(zin, Ws[l], bm=1024)
        parts = edge_call(H.reshape(num_rel * npad, hid), gidx, srcp)
        z = bn_call(parts, gs[l].reshape(1, hid), bs[l].reshape(1, hid))
        zs.append(z[:n])
        if l < 2:
            zin = jnp.concatenate([x_pad[:, :id_dim], z], axis=1)

    z_cat = jnp.concatenate(zs, axis=1)
    g_cat = _make_seg(n, 3 * hid)(
        batch.reshape(1, n).astype(jnp.int32), z_cat)
    return (z_cat, g_cat)


# speedup vs baseline: 11.7097x; 1.0864x over previous
import functools

import jax
import jax.numpy as jnp
from jax import lax
from jax.experimental import pallas as pl
from jax.experimental.pallas import tpu as pltpu
from jax.experimental.pallas import tpu_sc as plsc

_NUM_GRAPHS = 16
_EPS = 1e-5
_NC = 2
_NS = 16
_CHUNK = 128


def _mm_body(z_ref, w_ref, h_ref):
    h_ref[0] = jnp.maximum(
        lax.dot_general(z_ref[...], w_ref[0], (((1,), (0,)), ((), ())),
                        preferred_element_type=jnp.float32),
        0.0)


def _relu_mm(zin, W, bm):
    npad, d = zin.shape
    num_rel, _, hid = W.shape
    return pl.pallas_call(
        _mm_body,
        grid=(npad // bm, num_rel),
        in_specs=[
            pl.BlockSpec((bm, d), lambda i, r: (i, 0)),
            pl.BlockSpec((1, d, hid), lambda i, r: (r, 0, 0)),
        ],
        out_specs=pl.BlockSpec((1, bm, hid), lambda i, r: (r, i, 0)),
        out_shape=jax.ShapeDtypeStruct((num_rel, npad, hid), jnp.float32),
    )(zin, W)


def _make_edge_kernel(npad, hid, chunks_per_worker, nrows_tab):
    rows_per_tile = npad // _NS

    mesh = plsc.VectorSubcoreMesh(core_axis_name="c", subcore_axis_name="s")

    half = chunks_per_worker // 2

    @functools.partial(
        pl.kernel, mesh=mesh,
        out_type=jax.ShapeDtypeStruct((_NC, npad, hid), jnp.float32),
        scratch_types=[
            pltpu.VMEM((half, _CHUNK), jnp.int32),
            pltpu.VMEM((half, _CHUNK), jnp.int32),
            pltpu.VMEM((_CHUNK, hid), jnp.float32),
            pltpu.VMEM((_CHUNK, hid), jnp.float32),
            pltpu.VMEM_SHARED((npad, hid), jnp.float32),
            pltpu.SemaphoreType.DMA,
            pltpu.SemaphoreType.DMA,
        ],
    )
    def edge_kernel(h_hbm, gidx_hbm, src_hbm, out_hbm, gidx_v, src_v, rows_a,
                    rows_b, acc_sh, sem0, sem1):
        c = lax.axis_index("c")
        s = lax.axis_index("s")
        wid = c * _NS + s
        row_base = wid * chunks_per_worker

        def _zero_body(i, carry):
            r = i // (hid // 16)
            col = (i % (hid // 16)) * 16
            rows_a[r, pl.ds(col, 16)] = jnp.zeros((16,), jnp.float32)
            return carry
        lax.fori_loop(0, _CHUNK * (hid // 16), _zero_body, 0)
        for t in range(rows_per_tile // _CHUNK):
            pltpu.sync_copy(
                rows_a,
                acc_sh.at[pl.ds(s * rows_per_tile + t * _CHUNK, _CHUNK)])
        plsc.subcore_barrier()

        for h in range(2):
            pltpu.sync_copy(
                gidx_hbm.at[pl.ds(row_base + h * half, half)], gidx_v)
            pltpu.sync_copy(
                src_hbm.at[pl.ds(row_base + h * half, half)], src_v)
            pltpu.async_copy(h_hbm.at[gidx_v.at[0]], rows_a, sem0)

            def _body(i, carry):
                k0 = 2 * i
                pltpu.async_copy(h_hbm.at[gidx_v.at[k0 + 1]], rows_b, sem1)
                pltpu.make_async_copy(
                    h_hbm.at[pl.ds(0, _CHUNK)], rows_a, sem0).wait()
                pltpu.sync_copy(rows_a, acc_sh.at[src_v.at[k0]], add=True)
                k2 = jnp.minimum(k0 + 2, half - 2)
                pltpu.async_copy(h_hbm.at[gidx_v.at[k2]], rows_a, sem0)
                pltpu.make_async_copy(
                    h_hbm.at[pl.ds(0, _CHUNK)], rows_b, sem1).wait()
                pltpu.sync_copy(rows_b, acc_sh.at[src_v.at[k0 + 1]],
                                add=True)
                return carry
            lax.fori_loop(0, half // 2, _body, 0)
            pltpu.make_async_copy(
                h_hbm.at[pl.ds(0, _CHUNK)], rows_a, sem0).wait()
        plsc.subcore_barrier()

        for t in range(rows_per_tile // _CHUNK):
            off = s * rows_per_tile + t * _CHUNK
            pltpu.sync_copy(acc_sh.at[pl.ds(off, _CHUNK)], rows_a)
            pltpu.sync_copy(rows_a, out_hbm.at[c, pl.ds(off, _CHUNK)])

    return edge_kernel


def _make_bn(n, npad, hid):
    def _bn_body(p_ref, g_ref, b_ref, z_ref):
        o = p_ref[0, :n, :] + p_ref[1, :n, :]
        mean = jnp.mean(o, axis=0, keepdims=True)
        d = o - mean
        var = jnp.mean(d * d, axis=0, keepdims=True)
        zn = d * lax.rsqrt(var + _EPS) * g_ref[...] + b_ref[...]
        z_ref[:n, :] = jnp.maximum(zn, 0.0)
        z_ref[n:, :] = jnp.zeros((npad - n, hid), jnp.float32)

    return pl.pallas_call(
        _bn_body,
        out_shape=jax.ShapeDtypeStruct((npad, hid), jnp.float32),
    )


def _make_seg(n, dim):
    def _seg_body(b_ref, z_ref, g_ref):
        ids = jnp.broadcast_to(b_ref[...], (_NUM_GRAPHS, n))
        onehot = (ids == lax.broadcasted_iota(jnp.int32, (_NUM_GRAPHS, n), 0)
                  ).astype(jnp.float32)
        g_ref[...] = lax.dot_general(
            onehot, z_ref[...], (((1,), (0,)), ((), ())),
            preferred_element_type=jnp.float32)

    return pl.pallas_call(
        _seg_body,
        out_shape=jax.ShapeDtypeStruct((_NUM_GRAPHS, dim), jnp.float32),
    )


def kernel(x, edge_index, edge_type, batch, W0, W1, W2, g0, b0, g1, b1, g2,
           b2):
    n, in_dim = x.shape
    ne = edge_index.shape[1]
    num_rel, _, hid = W0.shape
    id_dim = W1.shape[1] - hid

    nw = _NC * _NS
    npad = ((n + 2047) // 2048) * 2048
    epw = ((ne + nw * _CHUNK * 8 - 1) // (nw * _CHUNK * 8)) * _CHUNK * 8
    ne_pad = epw * nw
    chunks_per_worker = epw // _CHUNK

    src = edge_index[0].astype(jnp.int32)
    dst = edge_index[1].astype(jnp.int32)
    et = edge_type.astype(jnp.int32)
    pad = ne_pad - ne
    gidx = jnp.concatenate(
        [et * npad + dst, jnp.zeros((pad,), jnp.int32)]).reshape(-1, _CHUNK)
    srcp = jnp.concatenate(
        [src, jnp.full((pad,), npad - 1, jnp.int32)]).reshape(-1, _CHUNK)

    x_pad = jnp.pad(x, ((0, npad - n), (0, 0)))
    edge_call = _make_edge_kernel(npad, hid, chunks_per_worker,
                                  num_rel * npad)
    bn_call = _make_bn(n, npad, hid)

    Ws = (W0, W1, W2)
    gs = (g0, g1, g2)
    bs = (b0, b1, b2)
    zin = x_pad
    zs = []
    for l in range(3):
        H = _relu_mm(zin, Ws[l], bm=1024)
        parts = edge_call(H.reshape(num_rel * npad, hid), gidx, srcp)
        z = bn_call(parts, gs[l].reshape(1, hid), bs[l].reshape(1, hid))
        zs.append(z[:n])
        if l < 2:
            zin = jnp.concatenate([x_pad[:, :id_dim], z], axis=1)

    z_cat = jnp.concatenate(zs, axis=1)
    g_cat = _make_seg(n, 3 * hid)(
        batch.reshape(1, n).astype(jnp.int32), z_cat)
    return (z_cat, g_cat)
